# baseline (speedup 1.0000x reference)
import functools

import jax
import jax.numpy as jnp
from jax import lax
from jax.experimental import pallas as pl
from jax.experimental.pallas import tpu as pltpu
from jax.experimental.pallas import tpu_sc as plsc

NC, NS, L = 2, 16, 16
NW = NC * NS
C = 128
N_PARTS = 20
D = 64
CB = 512


def _repack_body(x1_ref, x2_ref, o_ref):
    o_ref[:, 0:D] = jnp.transpose(x1_ref[...])
    o_ref[:, D:2 * D] = jnp.transpose(x2_ref[...])


def _repack(otabT, V):
    n_blocks = (V + CB - 1) // CB
    n_pair_rows = n_blocks * (CB // 2)
    max_bc = (V + CB // 2 - 1) // (CB // 2) - 1
    return pl.pallas_call(
        _repack_body,
        grid=(n_blocks,),
        in_specs=[
            pl.BlockSpec((D, CB // 2), lambda i: (0, 2 * i)),
            pl.BlockSpec((D, CB // 2),
                         lambda i: (0, jnp.minimum(2 * i + 1, max_bc))),
        ],
        out_specs=pl.BlockSpec((CB // 2, 2 * D), lambda i: (i, 0)),
        out_shape=jax.ShapeDtypeStruct((n_pair_rows, 2 * D), jnp.float32),
    )(otabT, otabT)


def _sc_body(tpr, opr, spr, tpairs, opairs, spairs,
             t_out, o_out, s_out,
             tidx_v, oidx_v, sidx_v, trows, orows, srows,
             sem_t, sem_o, sem_s):
    wid = lax.axis_index("s") * NC + lax.axis_index("c")
    rows_per_worker = tpr.shape[0] // NW
    n_chunks = rows_per_worker // C

    for c in range(n_chunks):
        base = wid * rows_per_worker + c * C

        pltpu.sync_copy(tpr.at[pl.ds(base, C)], tidx_v)
        pltpu.sync_copy(opr.at[pl.ds(base, C)], oidx_v)
        pltpu.sync_copy(spr.at[pl.ds(base, C)], sidx_v)

        cp_o = pltpu.async_copy(opairs.at[oidx_v], orows, sem_o)
        cp_t = pltpu.async_copy(tpairs.at[tidx_v], trows, sem_t)
        cp_s = pltpu.async_copy(spairs.at[sidx_v], srows, sem_s)
        cp_o.wait()
        cp_t.wait()
        cp_s.wait()

        pltpu.sync_copy(trows, t_out.at[pl.ds(base, C)])
        pltpu.sync_copy(orows, o_out.at[pl.ds(base, C)])
        pltpu.sync_copy(srows, s_out.at[pl.ds(base, C)])


def _sc_gather(tpr, opr, spr, tpairs, opairs, spairs, B):
    mesh = plsc.VectorSubcoreMesh(core_axis_name="c", subcore_axis_name="s")
    pr = jax.ShapeDtypeStruct((B, 2 * D), jnp.float32)
    f = pl.kernel(
        _sc_body,
        out_type=(pr, pr, pr),
        mesh=mesh,
        scratch_types=[
            pltpu.VMEM((C,), jnp.int32),
            pltpu.VMEM((C,), jnp.int32),
            pltpu.VMEM((C,), jnp.int32),
            pltpu.VMEM((C, 2 * D), jnp.float32),
            pltpu.VMEM((C, 2 * D), jnp.float32),
            pltpu.VMEM((C, 2 * D), jnp.float32),
            pltpu.SemaphoreType.DMA,
            pltpu.SemaphoreType.DMA,
            pltpu.SemaphoreType.DMA,
        ],
    )
    return f(tpr, opr, spr, tpairs, opairs, spairs)


def _finish_body(t_ref, o_ref2, s_ref, th_ref, oh_ref, sh_ref,
                 p_ref, w_ref, b_ref, out_ref):
    t = t_ref[...]
    o = o_ref2[...]
    s = s_ref[...]
    out_ref[:, 0:D] = jnp.where(th_ref[...] > 0.5, t[:, D:2 * D], t[:, 0:D])
    out_ref[:, D:2 * D] = jnp.where(oh_ref[...] > 0.5, o[:, D:2 * D], o[:, 0:D])
    out_ref[:, 2 * D:3 * D] = jnp.where(sh_ref[...] > 0.5, s[:, D:2 * D], s[:, 0:D])

    p = p_ref[...]
    w0 = w_ref[0:1, :]
    w1 = w_ref[1:2, :]
    acc = None
    for j in range(N_PARTS):
        p0 = p[:, 2 * j:2 * j + 1]
        p1 = p[:, 2 * j + 1:2 * j + 2]
        v = p0 * w0 + p1 * w1
        acc = v if acc is None else jnp.maximum(acc, v)
    out_ref[:, 3 * D:4 * D] = acc + b_ref[...]


def _finish(tg, og, sg, th, oh, sh, pflat, part_W, part_b, B):
    BB = 512
    vec = pl.BlockSpec((BB, 2 * D), lambda i: (i, 0))
    col = pl.BlockSpec((BB, 1), lambda i: (i, 0))
    return pl.pallas_call(
        _finish_body,
        grid=(B // BB,),
        in_specs=[
            vec, vec, vec, col, col, col,
            pl.BlockSpec((BB, 2 * N_PARTS), lambda i: (i, 0)),
            pl.BlockSpec((2, D), lambda i: (0, 0)),
            pl.BlockSpec((1, D), lambda i: (0, 0)),
        ],
        out_specs=pl.BlockSpec((BB, 4 * D), lambda i: (i, 0)),
        out_shape=jax.ShapeDtypeStruct((B, 4 * D), jnp.float32),
    )(tg, og, sg, th, oh, sh, pflat, part_W, part_b.reshape(1, D))


def kernel(tasks, object_classes, states, parts, task_table, object_table,
           state_table, part_W, part_b):
    B = parts.shape[0]
    V = object_table.shape[0]
    pflat = parts.reshape(B, 2 * N_PARTS).astype(jnp.float32)
    tidx = tasks.astype(jnp.int32)
    oidx = object_classes.astype(jnp.int32)
    sidx = states.astype(jnp.int32)

    opairs = _repack(object_table.T, V)
    tpairs = task_table.reshape(task_table.shape[0] // 2, 2 * D)
    spairs = state_table.reshape(state_table.shape[0] // 2, 2 * D)

    opr = (oidx >> 9) * 256 + (oidx & 255)
    oh = ((oidx >> 8) & 1).astype(jnp.float32).reshape(B, 1)
    tpr = tidx >> 1
    th = (tidx & 1).astype(jnp.float32).reshape(B, 1)
    spr = sidx >> 1
    sh = (sidx & 1).astype(jnp.float32).reshape(B, 1)

    tg, og, sg = _sc_gather(tpr, opr, spr, tpairs, opairs, spairs, B)
    return _finish(tg, og, sg, th, oh, sh, pflat, part_W, part_b, B)


# speedup vs baseline: 2.1420x; 2.1420x over previous
import functools

import jax
import jax.numpy as jnp
from jax import lax
from jax.experimental import pallas as pl
from jax.experimental.pallas import tpu as pltpu
from jax.experimental.pallas import tpu_sc as plsc

NC, NS, L = 2, 16, 16
NW = NC * NS
C = 128
N_PARTS = 20
D = 64
CB = 2048


def _repack_body(x1_ref, x2_ref, eye_ref, o_ref):
    eye = eye_ref[...]
    dn = (((0,), (0,)), ((), ()))
    o_ref[:, 0:D] = lax.dot_general(x1_ref[...], eye, dn,
                                    preferred_element_type=jnp.float32)
    o_ref[:, D:2 * D] = lax.dot_general(x2_ref[...], eye, dn,
                                        preferred_element_type=jnp.float32)


def _repack(otabT, V):
    n_blocks = (V + CB - 1) // CB
    n_pair_rows = n_blocks * (CB // 2)
    max_bc = (V + CB // 2 - 1) // (CB // 2) - 1
    eye = jnp.eye(D, dtype=jnp.float32)
    return pl.pallas_call(
        _repack_body,
        grid=(n_blocks,),
        in_specs=[
            pl.BlockSpec((D, CB // 2), lambda i: (0, 2 * i)),
            pl.BlockSpec((D, CB // 2),
                         lambda i: (0, jnp.minimum(2 * i + 1, max_bc))),
            pl.BlockSpec((D, D), lambda i: (0, 0)),
        ],
        out_specs=pl.BlockSpec((CB // 2, 2 * D), lambda i: (i, 0)),
        out_shape=jax.ShapeDtypeStruct((n_pair_rows, 2 * D), jnp.float32),
    )(otabT, otabT, eye)


def _sc_body(tpr, opr, spr, tpairs, opairs, spairs,
             t_out, o_out, s_out,
             tidx_v, oidx_v, sidx_v, trows, orows, srows,
             sem_t, sem_o, sem_s):
    wid = lax.axis_index("s") * NC + lax.axis_index("c")
    rows_per_worker = tpr.shape[0] // NW
    n_chunks = rows_per_worker // C

    for c in range(n_chunks):
        base = wid * rows_per_worker + c * C

        pltpu.sync_copy(tpr.at[pl.ds(base, C)], tidx_v)
        pltpu.sync_copy(opr.at[pl.ds(base, C)], oidx_v)
        pltpu.sync_copy(spr.at[pl.ds(base, C)], sidx_v)

        cp_o = pltpu.async_copy(opairs.at[oidx_v], orows, sem_o)
        cp_t = pltpu.async_copy(tpairs.at[tidx_v], trows, sem_t)
        cp_s = pltpu.async_copy(spairs.at[sidx_v], srows, sem_s)
        cp_o.wait()
        cp_t.wait()
        cp_s.wait()

        pltpu.sync_copy(trows, t_out.at[pl.ds(base, C)])
        pltpu.sync_copy(orows, o_out.at[pl.ds(base, C)])
        pltpu.sync_copy(srows, s_out.at[pl.ds(base, C)])


def _sc_gather(tpr, opr, spr, tpairs, opairs, spairs, B):
    mesh = plsc.VectorSubcoreMesh(core_axis_name="c", subcore_axis_name="s")
    pr = jax.ShapeDtypeStruct((B, 2 * D), jnp.float32)
    f = pl.kernel(
        _sc_body,
        out_type=(pr, pr, pr),
        mesh=mesh,
        scratch_types=[
            pltpu.VMEM((C,), jnp.int32),
            pltpu.VMEM((C,), jnp.int32),
            pltpu.VMEM((C,), jnp.int32),
            pltpu.VMEM((C, 2 * D), jnp.float32),
            pltpu.VMEM((C, 2 * D), jnp.float32),
            pltpu.VMEM((C, 2 * D), jnp.float32),
            pltpu.SemaphoreType.DMA,
            pltpu.SemaphoreType.DMA,
            pltpu.SemaphoreType.DMA,
        ],
    )
    return f(tpr, opr, spr, tpairs, opairs, spairs)


def _finish_body(t_ref, o_ref2, s_ref, th_ref, oh_ref, sh_ref,
                 p_ref, w_ref, b_ref, out_ref):
    t = t_ref[...]
    o = o_ref2[...]
    s = s_ref[...]
    out_ref[:, 0:D] = jnp.where(th_ref[...] > 0.5, t[:, D:2 * D], t[:, 0:D])
    out_ref[:, D:2 * D] = jnp.where(oh_ref[...] > 0.5, o[:, D:2 * D], o[:, 0:D])
    out_ref[:, 2 * D:3 * D] = jnp.where(sh_ref[...] > 0.5, s[:, D:2 * D], s[:, 0:D])

    p = p_ref[...]
    w0 = w_ref[0:1, :]
    w1 = w_ref[1:2, :]
    acc = None
    for j in range(N_PARTS):
        p0 = p[:, 2 * j:2 * j + 1]
        p1 = p[:, 2 * j + 1:2 * j + 2]
        v = p0 * w0 + p1 * w1
        acc = v if acc is None else jnp.maximum(acc, v)
    out_ref[:, 3 * D:4 * D] = acc + b_ref[...]


def _finish(tg, og, sg, th, oh, sh, pflat, part_W, part_b, B):
    BB = 512
    vec = pl.BlockSpec((BB, 2 * D), lambda i: (i, 0))
    col = pl.BlockSpec((BB, 1), lambda i: (i, 0))
    return pl.pallas_call(
        _finish_body,
        grid=(B // BB,),
        in_specs=[
            vec, vec, vec, col, col, col,
            pl.BlockSpec((BB, 2 * N_PARTS), lambda i: (i, 0)),
            pl.BlockSpec((2, D), lambda i: (0, 0)),
            pl.BlockSpec((1, D), lambda i: (0, 0)),
        ],
        out_specs=pl.BlockSpec((BB, 4 * D), lambda i: (i, 0)),
        out_shape=jax.ShapeDtypeStruct((B, 4 * D), jnp.float32),
    )(tg, og, sg, th, oh, sh, pflat, part_W, part_b.reshape(1, D))


def kernel(tasks, object_classes, states, parts, task_table, object_table,
           state_table, part_W, part_b):
    B = parts.shape[0]
    V = object_table.shape[0]
    pflat = parts.reshape(B, 2 * N_PARTS).astype(jnp.float32)
    tidx = tasks.astype(jnp.int32)
    oidx = object_classes.astype(jnp.int32)
    sidx = states.astype(jnp.int32)

    opairs = _repack(object_table.T, V)
    tpairs = task_table.reshape(task_table.shape[0] // 2, 2 * D)
    spairs = state_table.reshape(state_table.shape[0] // 2, 2 * D)

    opr = (oidx >> 11) * (CB // 2) + (oidx & (CB // 2 - 1))
    oh = ((oidx >> 10) & 1).astype(jnp.float32).reshape(B, 1)
    tpr = tidx >> 1
    th = (tidx & 1).astype(jnp.float32).reshape(B, 1)
    spr = sidx >> 1
    sh = (sidx & 1).astype(jnp.float32).reshape(B, 1)

    tg, og, sg = _sc_gather(tpr, opr, spr, tpairs, opairs, spairs, B)
    return _finish(tg, og, sg, th, oh, sh, pflat, part_W, part_b, B)
